# baseline (speedup 1.0000x reference)
import functools

import jax
import jax.numpy as jnp
from jax import lax
from jax.experimental import pallas as pl
from jax.experimental.pallas import tpu as pltpu
from jax.experimental.pallas import tpu_sc as plsc

_N = 10000
_E = 320000
_D = 128
_H = 256
_OUT = 128
_NS = 16
_K = 128
_CPT = 160
_EPT = _CPT * _K
_E2 = _NS * _EPT
_CPT_DEG = 80
_EPT_DEG = _CPT_DEG * _K
_E2D = 2 * _NS * _EPT_DEG
_G = 80
_NG = _N // _G
_GPT = 8
_NB = 2000


def _row_groups(s, copy_one):
    for g in range(_GPT):
        grp = s + _NS * g

        @pl.when(grp < _NG)
        def _(grp=grp):
            copy_one(grp * _G)


def _deg_body(dstd_hbm, ones_hbm, zeros_hbm, deg_hbm, dsts_i, ones_v, acc_sh,
              sem_a, sem_b):
    c = lax.axis_index("c")
    s = lax.axis_index("s")
    _row_groups(s, lambda r0: pltpu.sync_copy(
        zeros_hbm.at[pl.ds(r0, _G)], acc_sh.at[pl.ds(r0, _G)]))
    pltpu.sync_copy(ones_hbm, ones_v)
    rbase = (c * _NS + s) * _CPT_DEG
    pltpu.sync_copy(dstd_hbm.at[pl.ds(rbase, _CPT_DEG)], dsts_i)
    plsc.subcore_barrier()

    def body(j, carry):
        a = pltpu.async_copy(ones_v, acc_sh.at[dsts_i.at[2 * j]], sem_a,
                             add=True)
        b = pltpu.async_copy(ones_v, acc_sh.at[dsts_i.at[2 * j + 1]], sem_b,
                             add=True)
        a.wait()
        b.wait()
        return carry

    lax.fori_loop(0, _CPT_DEG // 2, body, 0)
    plsc.subcore_barrier()
    _row_groups(s, lambda r0: pltpu.sync_copy(
        acc_sh.at[pl.ds(r0, _G)], deg_hbm.at[pl.ds(c * _N + r0, _G)]))


@functools.cache
def _deg_kernel():
    return pl.kernel(
        _deg_body,
        out_type=jax.ShapeDtypeStruct((2 * _N, _D), jnp.float32),
        mesh=plsc.VectorSubcoreMesh(core_axis_name="c", subcore_axis_name="s",
                                    num_cores=2, num_subcores=_NS),
        scratch_types=[
            pltpu.VMEM((_CPT_DEG, _K), jnp.int32),
            pltpu.VMEM((_K, _D), jnp.float32),
            pltpu.MemorySpace.VMEM_SHARED((_N + 8, _D), jnp.float32),
            pltpu.SemaphoreType.DMA,
            pltpu.SemaphoreType.DMA,
        ],
    )


def _run_deg(dstd, ones16, zeros16):
    return _deg_kernel()(dstd, ones16, zeros16)


def _agg_body(y_hbm, src_hbm, dst_hbm, z_hbm, src_v, dst_v, rows_v, acc_sh, sem):
    c = lax.axis_index("c")
    s = lax.axis_index("s")
    _row_groups(s, lambda r0: pltpu.sync_copy(
        y_hbm.at[pl.ds(c * _N + r0, _G)], acc_sh.at[pl.ds(r0, _G)]))
    plsc.subcore_barrier()
    ebase = c * _E2 + s * _EPT
    dbase = s * _EPT

    def body(i, carry):
        off = i * _K
        pltpu.sync_copy(src_hbm.at[pl.ds(ebase + off, _K)], src_v)
        pltpu.sync_copy(dst_hbm.at[pl.ds(dbase + off, _K)], dst_v)
        pltpu.async_copy(y_hbm.at[src_v], rows_v, sem).wait()
        pltpu.sync_copy(rows_v, acc_sh.at[dst_v], add=True)
        return carry

    lax.fori_loop(0, _EPT // _K, body, 0)
    plsc.subcore_barrier()
    _row_groups(s, lambda r0: pltpu.sync_copy(
        acc_sh.at[pl.ds(r0, _G)], z_hbm.at[pl.ds(c * _N + r0, _G)]))


@functools.cache
def _agg_kernel():
    return pl.kernel(
        _agg_body,
        out_type=jax.ShapeDtypeStruct((2 * _N, _D), jnp.float32),
        mesh=plsc.VectorSubcoreMesh(core_axis_name="c", subcore_axis_name="s",
                                    num_cores=2, num_subcores=_NS),
        scratch_types=[
            pltpu.VMEM((_K,), jnp.int32),
            pltpu.VMEM((_K,), jnp.int32),
            pltpu.VMEM((_K, _D), jnp.float32),
            pltpu.MemorySpace.VMEM_SHARED((_N + 8, _D), jnp.float32),
            pltpu.SemaphoreType.DMA,
        ],
    )


def _run_agg(y, srcp, dstp):
    return _agg_kernel()(y, srcp, dstp)


def _dinv_block(deg_ref):
    deg = deg_ref[0, :, 0] + deg_ref[1, :, 0] + 1.0
    return (1.0 / jnp.sqrt(deg))[:, None]


def _lin1_body(deg_ref, x1_ref, x2_ref, w_ref, y_ref):
    w = w_ref[...]
    xw = jnp.dot(x1_ref[...], w[:_D, :], preferred_element_type=jnp.float32)
    xw = xw + jnp.dot(x2_ref[...], w[_D:, :], preferred_element_type=jnp.float32)
    y_ref[...] = xw * _dinv_block(deg_ref)


def _mid_body(deg_ref, z_ref, b_ref, w_ref, y_ref):
    dinv = _dinv_block(deg_ref)
    z = jnp.concatenate([z_ref[0], z_ref[1]], axis=-1)
    h = jnp.maximum(z * dinv + b_ref[...], 0.0)
    y_ref[...] = jnp.dot(h, w_ref[...], preferred_element_type=jnp.float32) * dinv


def _out_body(deg_ref, z_ref, b_ref, wfc_ref, bfc_ref, o_ref):
    dinv = _dinv_block(deg_ref)
    z = jnp.concatenate([z_ref[0], z_ref[1]], axis=-1)
    h = jnp.maximum(z * dinv + b_ref[...], 0.0)
    o_ref[...] = (jnp.dot(h, wfc_ref[...], preferred_element_type=jnp.float32)
                  + bfc_ref[...])


_NRB = _N // _NB


def _lin1(deg2, x1, x2, W1):
    return pl.pallas_call(
        _lin1_body,
        grid=(2, _NRB),
        in_specs=[
            pl.BlockSpec((2, _NB, _D), lambda c, r: (0, r, 0)),
            pl.BlockSpec((_NB, _D), lambda c, r: (r, 0)),
            pl.BlockSpec((_NB, _D), lambda c, r: (r, 0)),
            pl.BlockSpec((_H, _D), lambda c, r: (0, c)),
        ],
        out_specs=pl.BlockSpec((_NB, _D), lambda c, r: (c * _NRB + r, 0)),
        out_shape=jax.ShapeDtypeStruct((2 * _N, _D), jnp.float32),
    )(deg2, x1, x2, W1)


def _mid(deg2, z, b1, W2):
    return pl.pallas_call(
        _mid_body,
        grid=(2, _NRB),
        in_specs=[
            pl.BlockSpec((2, _NB, _D), lambda c, r: (0, r, 0)),
            pl.BlockSpec((2, _NB, _D), lambda c, r: (0, r, 0)),
            pl.BlockSpec((1, _H), lambda c, r: (0, 0)),
            pl.BlockSpec((_H, _D), lambda c, r: (0, c)),
        ],
        out_specs=pl.BlockSpec((_NB, _D), lambda c, r: (c * _NRB + r, 0)),
        out_shape=jax.ShapeDtypeStruct((2 * _N, _D), jnp.float32),
    )(deg2, z, b1, W2)


def _fc(deg2, z, b2, Wfc, bfc):
    return pl.pallas_call(
        _out_body,
        grid=(_NRB,),
        in_specs=[
            pl.BlockSpec((2, _NB, _D), lambda r: (0, r, 0)),
            pl.BlockSpec((2, _NB, _D), lambda r: (0, r, 0)),
            pl.BlockSpec((1, _H), lambda r: (0, 0)),
            pl.BlockSpec((_H, _OUT), lambda r: (0, 0)),
            pl.BlockSpec((1, _OUT), lambda r: (0, 0)),
        ],
        out_specs=pl.BlockSpec((_NB, _OUT), lambda r: (r, 0)),
        out_shape=jax.ShapeDtypeStruct((_N, _OUT), jnp.float32),
    )(deg2, z, b2, Wfc, bfc)


def kernel(input1, input2, edge_index, W1, b1, W2, b2, Wfc, bfc):
    src = edge_index[0]
    dst = edge_index[1]
    pad_a = _E2 - _E
    zpad = jnp.zeros((pad_a,), jnp.int32)
    npad = jnp.full((pad_a,), _N, jnp.int32)
    srcp = jnp.concatenate([src, zpad, src + _N, zpad])
    dstp = jnp.concatenate([dst, npad])
    ones128 = jnp.ones((_K, _D), jnp.float32)
    zeros128 = jnp.zeros((_N, _D), jnp.float32)

    deg2 = _run_deg(dstp.reshape(-1, _K), ones128,
                    zeros128).reshape(2, _N, _D)
    y1 = _lin1(deg2, input1, input2, W1)
    z1 = _run_agg(y1, srcp, dstp).reshape(2, _N, _D)
    y2 = _mid(deg2, z1, b1.reshape(1, _H), W2)
    z2 = _run_agg(y2, srcp, dstp).reshape(2, _N, _D)
    return _fc(deg2, z2, b2.reshape(1, _H), Wfc, bfc.reshape(1, _OUT))


# speedup vs baseline: 1.5686x; 1.5686x over previous
import functools

import jax
import jax.numpy as jnp
from jax import lax
from jax.experimental import pallas as pl
from jax.experimental.pallas import tpu as pltpu
from jax.experimental.pallas import tpu_sc as plsc

_N = 10000
_E = 320000
_D = 128
_H = 256
_OUT = 128
_NS = 16
_K = 128
_EPT = 20096
_E2 = _NS * _EPT
_EPT_DEG = 10112
_E2D = 2 * _NS * _EPT_DEG
_G = 80
_NG = _N // _G
_GPT = 8
_NB = 2000


def _row_groups(s, copy_one):
    for g in range(_GPT):
        grp = s + _NS * g

        @pl.when(grp < _NG)
        def _(grp=grp):
            copy_one(grp * _G)


def _deg_body(dstd_hbm, ones_hbm, zeros_hbm, deg_hbm, dst_v, ones_v, acc_sh):
    c = lax.axis_index("c")
    s = lax.axis_index("s")
    _row_groups(s, lambda r0: pltpu.sync_copy(
        zeros_hbm.at[pl.ds(r0, _G)], acc_sh.at[pl.ds(r0, _G)]))
    pltpu.sync_copy(ones_hbm, ones_v)
    plsc.subcore_barrier()
    base = (c * _NS + s) * _EPT_DEG

    def body(i, carry):
        pltpu.sync_copy(dstd_hbm.at[pl.ds(base + i * _K, _K)], dst_v)
        pltpu.sync_copy(ones_v, acc_sh.at[dst_v], add=True)
        return carry

    lax.fori_loop(0, _EPT_DEG // _K, body, 0)
    plsc.subcore_barrier()
    _row_groups(s, lambda r0: pltpu.sync_copy(
        acc_sh.at[pl.ds(r0, _G)], deg_hbm.at[pl.ds(c * _N + r0, _G)]))


@functools.cache
def _deg_kernel():
    return pl.kernel(
        _deg_body,
        out_type=jax.ShapeDtypeStruct((2 * _N, _D), jnp.float32),
        mesh=plsc.VectorSubcoreMesh(core_axis_name="c", subcore_axis_name="s",
                                    num_cores=2, num_subcores=_NS),
        scratch_types=[
            pltpu.VMEM((_K,), jnp.int32),
            pltpu.VMEM((_K, _D), jnp.float32),
            pltpu.MemorySpace.VMEM_SHARED((_N + 8, _D), jnp.float32),
        ],
    )


def _run_deg(dstd, ones16, zeros16):
    return _deg_kernel()(dstd, ones16, zeros16)


def _agg_body(y_hbm, src_hbm, dst_hbm, z_hbm, src_v, dst_v, rows_v, acc_sh, sem):
    c = lax.axis_index("c")
    s = lax.axis_index("s")
    _row_groups(s, lambda r0: pltpu.sync_copy(
        y_hbm.at[pl.ds(c * _N + r0, _G)], acc_sh.at[pl.ds(r0, _G)]))
    plsc.subcore_barrier()
    ebase = c * _E2 + s * _EPT
    dbase = s * _EPT

    def body(i, carry):
        off = i * _K
        pltpu.sync_copy(src_hbm.at[pl.ds(ebase + off, _K)], src_v)
        pltpu.sync_copy(dst_hbm.at[pl.ds(dbase + off, _K)], dst_v)
        pltpu.async_copy(y_hbm.at[src_v], rows_v, sem).wait()
        pltpu.sync_copy(rows_v, acc_sh.at[dst_v], add=True)
        return carry

    lax.fori_loop(0, _EPT // _K, body, 0)
    plsc.subcore_barrier()
    _row_groups(s, lambda r0: pltpu.sync_copy(
        acc_sh.at[pl.ds(r0, _G)], z_hbm.at[pl.ds(c * _N + r0, _G)]))


@functools.cache
def _agg_kernel():
    return pl.kernel(
        _agg_body,
        out_type=jax.ShapeDtypeStruct((2 * _N, _D), jnp.float32),
        mesh=plsc.VectorSubcoreMesh(core_axis_name="c", subcore_axis_name="s",
                                    num_cores=2, num_subcores=_NS),
        scratch_types=[
            pltpu.VMEM((_K,), jnp.int32),
            pltpu.VMEM((_K,), jnp.int32),
            pltpu.VMEM((_K, _D), jnp.float32),
            pltpu.MemorySpace.VMEM_SHARED((_N + 8, _D), jnp.float32),
            pltpu.SemaphoreType.DMA,
        ],
    )


def _run_agg(y, srcp, dstp):
    return _agg_kernel()(y, srcp, dstp)


def _dinv_block(deg_ref):
    deg = deg_ref[0, :, 0] + deg_ref[1, :, 0] + 1.0
    return (1.0 / jnp.sqrt(deg))[:, None]


def _lin1_body(deg_ref, x1_ref, x2_ref, w_ref, y_ref):
    w = w_ref[...]
    xw = jnp.dot(x1_ref[...], w[:_D, :], preferred_element_type=jnp.float32)
    xw = xw + jnp.dot(x2_ref[...], w[_D:, :], preferred_element_type=jnp.float32)
    y_ref[...] = xw * _dinv_block(deg_ref)


def _mid_body(deg_ref, z_ref, b_ref, w_ref, y_ref):
    dinv = _dinv_block(deg_ref)
    z = jnp.concatenate([z_ref[0], z_ref[1]], axis=-1)
    h = jnp.maximum(z * dinv + b_ref[...], 0.0)
    y_ref[...] = jnp.dot(h, w_ref[...], preferred_element_type=jnp.float32) * dinv


def _out_body(deg_ref, z_ref, b_ref, wfc_ref, bfc_ref, o_ref):
    dinv = _dinv_block(deg_ref)
    z = jnp.concatenate([z_ref[0], z_ref[1]], axis=-1)
    h = jnp.maximum(z * dinv + b_ref[...], 0.0)
    o_ref[...] = (jnp.dot(h, wfc_ref[...], preferred_element_type=jnp.float32)
                  + bfc_ref[...])


_NRB = _N // _NB


def _lin1(deg2, x1, x2, W1):
    return pl.pallas_call(
        _lin1_body,
        grid=(2, _NRB),
        in_specs=[
            pl.BlockSpec((2, _NB, _D), lambda c, r: (0, r, 0)),
            pl.BlockSpec((_NB, _D), lambda c, r: (r, 0)),
            pl.BlockSpec((_NB, _D), lambda c, r: (r, 0)),
            pl.BlockSpec((_H, _D), lambda c, r: (0, c)),
        ],
        out_specs=pl.BlockSpec((_NB, _D), lambda c, r: (c * _NRB + r, 0)),
        out_shape=jax.ShapeDtypeStruct((2 * _N, _D), jnp.float32),
    )(deg2, x1, x2, W1)


def _mid(deg2, z, b1, W2):
    return pl.pallas_call(
        _mid_body,
        grid=(2, _NRB),
        in_specs=[
            pl.BlockSpec((2, _NB, _D), lambda c, r: (0, r, 0)),
            pl.BlockSpec((2, _NB, _D), lambda c, r: (0, r, 0)),
            pl.BlockSpec((1, _H), lambda c, r: (0, 0)),
            pl.BlockSpec((_H, _D), lambda c, r: (0, c)),
        ],
        out_specs=pl.BlockSpec((_NB, _D), lambda c, r: (c * _NRB + r, 0)),
        out_shape=jax.ShapeDtypeStruct((2 * _N, _D), jnp.float32),
    )(deg2, z, b1, W2)


def _fc(deg2, z, b2, Wfc, bfc):
    return pl.pallas_call(
        _out_body,
        grid=(_NRB,),
        in_specs=[
            pl.BlockSpec((2, _NB, _D), lambda r: (0, r, 0)),
            pl.BlockSpec((2, _NB, _D), lambda r: (0, r, 0)),
            pl.BlockSpec((1, _H), lambda r: (0, 0)),
            pl.BlockSpec((_H, _OUT), lambda r: (0, 0)),
            pl.BlockSpec((1, _OUT), lambda r: (0, 0)),
        ],
        out_specs=pl.BlockSpec((_NB, _OUT), lambda r: (r, 0)),
        out_shape=jax.ShapeDtypeStruct((_N, _OUT), jnp.float32),
    )(deg2, z, b2, Wfc, bfc)


def kernel(input1, input2, edge_index, W1, b1, W2, b2, Wfc, bfc):
    src = edge_index[0]
    dst = edge_index[1]
    pad_a = _E2 - _E
    zpad = jnp.zeros((pad_a,), jnp.int32)
    srcp = jnp.concatenate([src, zpad, src + _N, zpad])
    dstp = jnp.concatenate([dst, jnp.full((pad_a,), _N, jnp.int32)])
    dstd = jnp.concatenate([dst, jnp.full((_E2D - _E,), _N, jnp.int32)])
    ones128 = jnp.ones((_K, _D), jnp.float32)
    zeros128 = jnp.zeros((_N, _D), jnp.float32)

    deg2 = _run_deg(dstd, ones128, zeros128).reshape(2, _N, _D)
    y1 = _lin1(deg2, input1, input2, W1)
    z1 = _run_agg(y1, srcp, dstp).reshape(2, _N, _D)
    y2 = _mid(deg2, z1, b1.reshape(1, _H), W2)
    z2 = _run_agg(y2, srcp, dstp).reshape(2, _N, _D)
    return _fc(deg2, z2, b2.reshape(1, _H), Wfc, bfc.reshape(1, _OUT))


# speedup vs baseline: 1.7385x; 1.1083x over previous
import functools

import jax
import jax.numpy as jnp
from jax import lax
from jax.experimental import pallas as pl
from jax.experimental.pallas import tpu as pltpu
from jax.experimental.pallas import tpu_sc as plsc

_N = 10000
_E = 320000
_D = 128
_H = 256
_OUT = 128
_NS = 16
_K = 128
_EPT = 20096
_E2 = _NS * _EPT
_EPT_DEG = 10112
_E2D = 2 * _NS * _EPT_DEG
_G = 80
_NG = _N // _G
_GPT = 8
_NB = 2000


def _row_groups(s, copy_one):
    for g in range(_GPT):
        grp = s + _NS * g

        @pl.when(grp < _NG)
        def _(grp=grp):
            copy_one(grp * _G)


def _deg_body(dstd_hbm, ones_hbm, zeros_hbm, deg_hbm, dst_v, ones_v, acc_sh):
    c = lax.axis_index("c")
    s = lax.axis_index("s")
    _row_groups(s, lambda r0: pltpu.sync_copy(
        zeros_hbm.at[pl.ds(r0, _G)], acc_sh.at[pl.ds(r0, _G)]))
    pltpu.sync_copy(ones_hbm, ones_v)
    plsc.subcore_barrier()
    base = (c * _NS + s) * _EPT_DEG

    def body(i, carry):
        pltpu.sync_copy(dstd_hbm.at[pl.ds(base + i * _K, _K)], dst_v)
        pltpu.sync_copy(ones_v, acc_sh.at[dst_v], add=True)
        return carry

    lax.fori_loop(0, _EPT_DEG // _K, body, 0)
    plsc.subcore_barrier()
    _row_groups(s, lambda r0: pltpu.sync_copy(
        acc_sh.at[pl.ds(r0, _G)], deg_hbm.at[pl.ds(c * _N + r0, _G)]))


@functools.cache
def _deg_kernel():
    return pl.kernel(
        _deg_body,
        out_type=jax.ShapeDtypeStruct((2 * _N, _D), jnp.float32),
        mesh=plsc.VectorSubcoreMesh(core_axis_name="c", subcore_axis_name="s",
                                    num_cores=2, num_subcores=_NS),
        scratch_types=[
            pltpu.VMEM((_K,), jnp.int32),
            pltpu.VMEM((_K, _D), jnp.float32),
            pltpu.MemorySpace.VMEM_SHARED((_N + 8, _D), jnp.float32),
        ],
    )


def _run_deg(dstd, ones16, zeros16):
    return _deg_kernel()(dstd, ones16, zeros16)


def _agg_body(y_hbm, src_hbm, dst_hbm, z_hbm, src_v, dst_v, rows_v, acc_sh, sem):
    c = lax.axis_index("c")
    s = lax.axis_index("s")
    _row_groups(s, lambda r0: pltpu.sync_copy(
        y_hbm.at[pl.ds(c * _N + r0, _G)], acc_sh.at[pl.ds(r0, _G)]))
    plsc.subcore_barrier()
    ebase = c * _E2 + s * _EPT
    dbase = s * _EPT

    def body(i, carry):
        off = i * _K
        pltpu.sync_copy(src_hbm.at[pl.ds(ebase + off, _K)], src_v)
        g = pltpu.async_copy(y_hbm.at[src_v], rows_v, sem)
        pltpu.sync_copy(dst_hbm.at[pl.ds(dbase + off, _K)], dst_v)
        g.wait()
        pltpu.sync_copy(rows_v, acc_sh.at[dst_v], add=True)
        return carry

    lax.fori_loop(0, _EPT // _K, body, 0)
    plsc.subcore_barrier()
    _row_groups(s, lambda r0: pltpu.sync_copy(
        acc_sh.at[pl.ds(r0, _G)], z_hbm.at[pl.ds(c * _N + r0, _G)]))


@functools.cache
def _agg_kernel():
    return pl.kernel(
        _agg_body,
        out_type=jax.ShapeDtypeStruct((2 * _N, _D), jnp.float32),
        mesh=plsc.VectorSubcoreMesh(core_axis_name="c", subcore_axis_name="s",
                                    num_cores=2, num_subcores=_NS),
        scratch_types=[
            pltpu.VMEM((_K,), jnp.int32),
            pltpu.VMEM((_K,), jnp.int32),
            pltpu.VMEM((_K, _D), jnp.float32),
            pltpu.MemorySpace.VMEM_SHARED((_N + 8, _D), jnp.float32),
            pltpu.SemaphoreType.DMA,
        ],
    )


def _run_agg(y, srcp, dstp):
    return _agg_kernel()(y, srcp, dstp)


def _dinv_block(deg_ref):
    deg = deg_ref[0, :, 0] + deg_ref[1, :, 0] + 1.0
    return (1.0 / jnp.sqrt(deg))[:, None]


def _lin1_body(deg_ref, x1_ref, x2_ref, w_ref, y_ref):
    w = w_ref[...]
    xw = jnp.dot(x1_ref[...], w[:_D, :], preferred_element_type=jnp.float32)
    xw = xw + jnp.dot(x2_ref[...], w[_D:, :], preferred_element_type=jnp.float32)
    y_ref[...] = xw * _dinv_block(deg_ref)


def _mid_body(deg_ref, z_ref, b_ref, w_ref, y_ref):
    dinv = _dinv_block(deg_ref)
    z = jnp.concatenate([z_ref[0], z_ref[1]], axis=-1)
    h = jnp.maximum(z * dinv + b_ref[...], 0.0)
    y_ref[...] = jnp.dot(h, w_ref[...], preferred_element_type=jnp.float32) * dinv


def _out_body(deg_ref, z_ref, b_ref, wfc_ref, bfc_ref, o_ref):
    dinv = _dinv_block(deg_ref)
    z = jnp.concatenate([z_ref[0], z_ref[1]], axis=-1)
    h = jnp.maximum(z * dinv + b_ref[...], 0.0)
    o_ref[...] = (jnp.dot(h, wfc_ref[...], preferred_element_type=jnp.float32)
                  + bfc_ref[...])


_NRB = _N // _NB


def _lin1(deg2, x1, x2, W1):
    return pl.pallas_call(
        _lin1_body,
        grid=(2, _NRB),
        in_specs=[
            pl.BlockSpec((2, _NB, _D), lambda c, r: (0, r, 0)),
            pl.BlockSpec((_NB, _D), lambda c, r: (r, 0)),
            pl.BlockSpec((_NB, _D), lambda c, r: (r, 0)),
            pl.BlockSpec((_H, _D), lambda c, r: (0, c)),
        ],
        out_specs=pl.BlockSpec((_NB, _D), lambda c, r: (c * _NRB + r, 0)),
        out_shape=jax.ShapeDtypeStruct((2 * _N, _D), jnp.float32),
    )(deg2, x1, x2, W1)


def _mid(deg2, z, b1, W2):
    return pl.pallas_call(
        _mid_body,
        grid=(2, _NRB),
        in_specs=[
            pl.BlockSpec((2, _NB, _D), lambda c, r: (0, r, 0)),
            pl.BlockSpec((2, _NB, _D), lambda c, r: (0, r, 0)),
            pl.BlockSpec((1, _H), lambda c, r: (0, 0)),
            pl.BlockSpec((_H, _D), lambda c, r: (0, c)),
        ],
        out_specs=pl.BlockSpec((_NB, _D), lambda c, r: (c * _NRB + r, 0)),
        out_shape=jax.ShapeDtypeStruct((2 * _N, _D), jnp.float32),
    )(deg2, z, b1, W2)


def _fc(deg2, z, b2, Wfc, bfc):
    return pl.pallas_call(
        _out_body,
        grid=(_NRB,),
        in_specs=[
            pl.BlockSpec((2, _NB, _D), lambda r: (0, r, 0)),
            pl.BlockSpec((2, _NB, _D), lambda r: (0, r, 0)),
            pl.BlockSpec((1, _H), lambda r: (0, 0)),
            pl.BlockSpec((_H, _OUT), lambda r: (0, 0)),
            pl.BlockSpec((1, _OUT), lambda r: (0, 0)),
        ],
        out_specs=pl.BlockSpec((_NB, _OUT), lambda r: (r, 0)),
        out_shape=jax.ShapeDtypeStruct((_N, _OUT), jnp.float32),
    )(deg2, z, b2, Wfc, bfc)


def kernel(input1, input2, edge_index, W1, b1, W2, b2, Wfc, bfc):
    src = edge_index[0]
    dst = edge_index[1]
    pad_a = _E2 - _E
    zpad = jnp.zeros((pad_a,), jnp.int32)
    srcp = jnp.concatenate([src, zpad, src + _N, zpad])
    dstp = jnp.concatenate([dst, jnp.full((pad_a,), _N, jnp.int32)])
    dstd = jnp.concatenate([dst, jnp.full((_E2D - _E,), _N, jnp.int32)])
    ones128 = jnp.ones((_K, _D), jnp.float32)
    zeros128 = jnp.zeros((_N, _D), jnp.float32)

    deg2 = _run_deg(dstd, ones128, zeros128).reshape(2, _N, _D)
    y1 = _lin1(deg2, input1, input2, W1)
    z1 = _run_agg(y1, srcp, dstp).reshape(2, _N, _D)
    y2 = _mid(deg2, z1, b1.reshape(1, _H), W2)
    z2 = _run_agg(y2, srcp, dstp).reshape(2, _N, _D)
    return _fc(deg2, z2, b2.reshape(1, _H), Wfc, bfc.reshape(1, _OUT))


# speedup vs baseline: 1.8809x; 1.0819x over previous
import functools

import jax
import jax.numpy as jnp
from jax import lax
from jax.experimental import pallas as pl
from jax.experimental.pallas import tpu as pltpu
from jax.experimental.pallas import tpu_sc as plsc

_N = 10000
_E = 320000
_D = 128
_H = 256
_OUT = 128
_NS = 16
_K = 128
_EPT = 20096
_E2 = _NS * _EPT
_EPT_DEG = 10112
_E2D = 2 * _NS * _EPT_DEG
_G = 80
_NG = _N // _G
_GPT = 8
_NB = 2000


def _row_groups(s, copy_one):
    for g in range(_GPT):
        grp = s + _NS * g

        @pl.when(grp < _NG)
        def _(grp=grp):
            copy_one(grp * _G)


def _deg_body(dstd_hbm, ones_hbm, zeros_hbm, deg_hbm, dst_v, ones_v, acc_sh):
    c = lax.axis_index("c")
    s = lax.axis_index("s")
    _row_groups(s, lambda r0: pltpu.sync_copy(
        zeros_hbm.at[pl.ds(r0, _G)], acc_sh.at[pl.ds(r0, _G)]))
    pltpu.sync_copy(ones_hbm, ones_v)
    plsc.subcore_barrier()
    base = (c * _NS + s) * _EPT_DEG

    def body(i, carry):
        pltpu.sync_copy(dstd_hbm.at[pl.ds(base + i * _K, _K)], dst_v)
        pltpu.sync_copy(ones_v, acc_sh.at[dst_v], add=True)
        return carry

    lax.fori_loop(0, _EPT_DEG // _K, body, 0)
    plsc.subcore_barrier()
    _row_groups(s, lambda r0: pltpu.sync_copy(
        acc_sh.at[pl.ds(r0, _G)], deg_hbm.at[pl.ds(c * _N + r0, _G)]))


@functools.cache
def _deg_kernel():
    return pl.kernel(
        _deg_body,
        out_type=jax.ShapeDtypeStruct((2 * _N, _D), jnp.float32),
        mesh=plsc.VectorSubcoreMesh(core_axis_name="c", subcore_axis_name="s",
                                    num_cores=2, num_subcores=_NS),
        scratch_types=[
            pltpu.VMEM((_K,), jnp.int32),
            pltpu.VMEM((_K, _D), jnp.float32),
            pltpu.MemorySpace.VMEM_SHARED((_N + 8, _D), jnp.float32),
        ],
    )


def _run_deg(dstd, ones16, zeros16):
    return _deg_kernel()(dstd, ones16, zeros16)


def _agg_body(y_hbm, src_hbm, dst_hbm, z_hbm, src_v, dst_v, src_b, dst_b,
              rows_v, rows_b, acc_sh, sem, ssem):
    c = lax.axis_index("c")
    s = lax.axis_index("s")
    _row_groups(s, lambda r0: pltpu.sync_copy(
        y_hbm.at[pl.ds(c * _N + r0, _G)], acc_sh.at[pl.ds(r0, _G)]))
    plsc.subcore_barrier()
    ebase = c * _E2 + s * _EPT
    dbase = s * _EPT

    def body(j, carry):
        off = 2 * j * _K
        pltpu.sync_copy(src_hbm.at[pl.ds(ebase + off, _K)], src_v)
        g = pltpu.async_copy(y_hbm.at[src_v], rows_v, sem)
        pltpu.sync_copy(dst_hbm.at[pl.ds(dbase + off, _K)], dst_v)
        g.wait()
        sa = pltpu.async_copy(rows_v, acc_sh.at[dst_v], ssem, add=True)
        pltpu.sync_copy(src_hbm.at[pl.ds(ebase + off + _K, _K)], src_b)
        g2 = pltpu.async_copy(y_hbm.at[src_b], rows_b, sem)
        pltpu.sync_copy(dst_hbm.at[pl.ds(dbase + off + _K, _K)], dst_b)
        g2.wait()
        sa.wait()
        pltpu.sync_copy(rows_b, acc_sh.at[dst_b], add=True)
        return carry

    lax.fori_loop(0, _EPT // (2 * _K), body, 0)
    off = _EPT - _K
    pltpu.sync_copy(src_hbm.at[pl.ds(ebase + off, _K)], src_v)
    g = pltpu.async_copy(y_hbm.at[src_v], rows_v, sem)
    pltpu.sync_copy(dst_hbm.at[pl.ds(dbase + off, _K)], dst_v)
    g.wait()
    pltpu.sync_copy(rows_v, acc_sh.at[dst_v], add=True)
    plsc.subcore_barrier()
    _row_groups(s, lambda r0: pltpu.sync_copy(
        acc_sh.at[pl.ds(r0, _G)], z_hbm.at[pl.ds(c * _N + r0, _G)]))


@functools.cache
def _agg_kernel():
    return pl.kernel(
        _agg_body,
        out_type=jax.ShapeDtypeStruct((2 * _N, _D), jnp.float32),
        mesh=plsc.VectorSubcoreMesh(core_axis_name="c", subcore_axis_name="s",
                                    num_cores=2, num_subcores=_NS),
        scratch_types=[
            pltpu.VMEM((_K,), jnp.int32),
            pltpu.VMEM((_K,), jnp.int32),
            pltpu.VMEM((_K,), jnp.int32),
            pltpu.VMEM((_K,), jnp.int32),
            pltpu.VMEM((_K, _D), jnp.float32),
            pltpu.VMEM((_K, _D), jnp.float32),
            pltpu.MemorySpace.VMEM_SHARED((_N + 8, _D), jnp.float32),
            pltpu.SemaphoreType.DMA,
            pltpu.SemaphoreType.DMA,
        ],
    )


def _run_agg(y, srcp, dstp):
    return _agg_kernel()(y, srcp, dstp)


def _dinv_block(deg_ref):
    deg = deg_ref[0, :, 0] + deg_ref[1, :, 0] + 1.0
    return (1.0 / jnp.sqrt(deg))[:, None]


def _lin1_body(deg_ref, x1_ref, x2_ref, w_ref, y_ref):
    w = w_ref[...]
    xw = jnp.dot(x1_ref[...], w[:_D, :], preferred_element_type=jnp.float32)
    xw = xw + jnp.dot(x2_ref[...], w[_D:, :], preferred_element_type=jnp.float32)
    y_ref[...] = xw * _dinv_block(deg_ref)


def _mid_body(deg_ref, z_ref, b_ref, w_ref, y_ref):
    dinv = _dinv_block(deg_ref)
    z = jnp.concatenate([z_ref[0], z_ref[1]], axis=-1)
    h = jnp.maximum(z * dinv + b_ref[...], 0.0)
    y_ref[...] = jnp.dot(h, w_ref[...], preferred_element_type=jnp.float32) * dinv


def _out_body(deg_ref, z_ref, b_ref, wfc_ref, bfc_ref, o_ref):
    dinv = _dinv_block(deg_ref)
    z = jnp.concatenate([z_ref[0], z_ref[1]], axis=-1)
    h = jnp.maximum(z * dinv + b_ref[...], 0.0)
    o_ref[...] = (jnp.dot(h, wfc_ref[...], preferred_element_type=jnp.float32)
                  + bfc_ref[...])


_NRB = _N // _NB


def _lin1(deg2, x1, x2, W1):
    return pl.pallas_call(
        _lin1_body,
        grid=(2, _NRB),
        in_specs=[
            pl.BlockSpec((2, _NB, _D), lambda c, r: (0, r, 0)),
            pl.BlockSpec((_NB, _D), lambda c, r: (r, 0)),
            pl.BlockSpec((_NB, _D), lambda c, r: (r, 0)),
            pl.BlockSpec((_H, _D), lambda c, r: (0, c)),
        ],
        out_specs=pl.BlockSpec((_NB, _D), lambda c, r: (c * _NRB + r, 0)),
        out_shape=jax.ShapeDtypeStruct((2 * _N, _D), jnp.float32),
    )(deg2, x1, x2, W1)


def _mid(deg2, z, b1, W2):
    return pl.pallas_call(
        _mid_body,
        grid=(2, _NRB),
        in_specs=[
            pl.BlockSpec((2, _NB, _D), lambda c, r: (0, r, 0)),
            pl.BlockSpec((2, _NB, _D), lambda c, r: (0, r, 0)),
            pl.BlockSpec((1, _H), lambda c, r: (0, 0)),
            pl.BlockSpec((_H, _D), lambda c, r: (0, c)),
        ],
        out_specs=pl.BlockSpec((_NB, _D), lambda c, r: (c * _NRB + r, 0)),
        out_shape=jax.ShapeDtypeStruct((2 * _N, _D), jnp.float32),
    )(deg2, z, b1, W2)


def _fc(deg2, z, b2, Wfc, bfc):
    return pl.pallas_call(
        _out_body,
        grid=(_NRB,),
        in_specs=[
            pl.BlockSpec((2, _NB, _D), lambda r: (0, r, 0)),
            pl.BlockSpec((2, _NB, _D), lambda r: (0, r, 0)),
            pl.BlockSpec((1, _H), lambda r: (0, 0)),
            pl.BlockSpec((_H, _OUT), lambda r: (0, 0)),
            pl.BlockSpec((1, _OUT), lambda r: (0, 0)),
        ],
        out_specs=pl.BlockSpec((_NB, _OUT), lambda r: (r, 0)),
        out_shape=jax.ShapeDtypeStruct((_N, _OUT), jnp.float32),
    )(deg2, z, b2, Wfc, bfc)


def kernel(input1, input2, edge_index, W1, b1, W2, b2, Wfc, bfc):
    src = edge_index[0]
    dst = edge_index[1]
    pad_a = _E2 - _E
    zpad = jnp.zeros((pad_a,), jnp.int32)
    srcp = jnp.concatenate([src, zpad, src + _N, zpad])
    dstp = jnp.concatenate([dst, jnp.full((pad_a,), _N, jnp.int32)])
    dstd = jnp.concatenate([dst, jnp.full((_E2D - _E,), _N, jnp.int32)])
    ones128 = jnp.ones((_K, _D), jnp.float32)
    zeros128 = jnp.zeros((_N, _D), jnp.float32)

    deg2 = _run_deg(dstd, ones128, zeros128).reshape(2, _N, _D)
    y1 = _lin1(deg2, input1, input2, W1)
    z1 = _run_agg(y1, srcp, dstp).reshape(2, _N, _D)
    y2 = _mid(deg2, z1, b1.reshape(1, _H), W2)
    z2 = _run_agg(y2, srcp, dstp).reshape(2, _N, _D)
    return _fc(deg2, z2, b2.reshape(1, _H), Wfc, bfc.reshape(1, _OUT))
